# Initial kernel scaffold; baseline (speedup 1.0000x reference)
import jax
import jax.numpy as jnp
from jax.experimental import pallas as pl


def kernel(input_ids, embedding, W_ih_f, W_hh_f, b_f, W_ih_b, W_hh_b, b_b, W_event, b_event, W_arg, b_arg):
    raise NotImplementedError("write your pallas kernel here")


# speedup vs baseline: 5.9019x; 5.9019x over previous
import functools

import jax
import jax.numpy as jnp
from jax import lax
from jax.experimental import pallas as pl
from jax.experimental.pallas import tpu as pltpu
from jax.experimental.pallas import tpu_sc as plsc

B = 32
L = 50
NE = 34
NA = 36
D = 128
H = 256
_F32 = jnp.float32


def _sc_gather(table, idx):
    info = plsc.get_sparse_core_info()
    nw = info.num_cores * info.num_subcores
    n = idx.shape[0]
    per = n // nw

    mesh = plsc.VectorSubcoreMesh(core_axis_name="c", subcore_axis_name="s")

    @functools.partial(
        pl.kernel,
        mesh=mesh,
        out_type=jax.ShapeDtypeStruct((n, D), _F32),
        scratch_types=[
            pltpu.VMEM((per,), jnp.int32),
            pltpu.VMEM((per, D), _F32),
            pltpu.SemaphoreType.DMA,
        ],
    )
    def gather_kernel(table_hbm, idx_hbm, out_hbm, idx_v, rows_v, sem):
        wid = lax.axis_index("s") * info.num_cores + lax.axis_index("c")
        base = wid * per
        pltpu.sync_copy(idx_hbm.at[pl.ds(base, per)], idx_v)
        pltpu.async_copy(table_hbm.at[idx_v], rows_v, sem).wait()
        pltpu.sync_copy(rows_v, out_hbm.at[pl.ds(base, per)])

    return gather_kernel(table, idx)


def _lstm_body(emb_ref, wif_ref, whf_ref, bf_ref, wib_ref, whb_ref, bb_ref,
               hf_ref, hb_ref, xf_ref, xb_ref):
    emb = emb_ref[...].reshape(L * B, D)
    xf_ref[...] = jnp.dot(
        emb, wif_ref[...], preferred_element_type=_F32).reshape(L, B, 4 * H)
    xb_ref[...] = jnp.dot(
        emb, wib_ref[...], preferred_element_type=_F32).reshape(L, B, 4 * H)
    whf = whf_ref[...]
    whb = whb_ref[...]
    bf = bf_ref[...]
    bb = bb_ref[...]
    zero = jnp.zeros((B, H), _F32)

    def cell(gates, c):
        i = jax.nn.sigmoid(gates[:, 0:H])
        f = jax.nn.sigmoid(gates[:, H:2 * H])
        g = jnp.tanh(gates[:, 2 * H:3 * H])
        o = jax.nn.sigmoid(gates[:, 3 * H:4 * H])
        c = f * c + i * g
        return o * jnp.tanh(c), c

    def fwd(t, carry):
        h, c = carry
        gates = (xf_ref[t] + jnp.dot(h, whf, preferred_element_type=_F32)) + bf
        h, c = cell(gates, c)
        hf_ref[t] = h
        return (h, c)

    def bwd(k, carry):
        t = L - 1 - k
        h, c = carry
        gates = (xb_ref[t] + jnp.dot(h, whb, preferred_element_type=_F32)) + bb
        h, c = cell(gates, c)
        hb_ref[t] = h
        return (h, c)

    lax.fori_loop(0, L, fwd, (zero, zero))
    lax.fori_loop(0, L, bwd, (zero, zero))


def _run_lstm(emb, wif_t, whf_t, bf, wib_t, whb_t, bb):
    return pl.pallas_call(
        _lstm_body,
        out_shape=[
            jax.ShapeDtypeStruct((L, B, H), _F32),
            jax.ShapeDtypeStruct((L, B, H), _F32),
        ],
        scratch_shapes=[
            pltpu.VMEM((L, B, 4 * H), _F32),
            pltpu.VMEM((L, B, 4 * H), _F32),
        ],
    )(emb, wif_t, whf_t, bf, wib_t, whb_t, bb)


def _decode_body(hf_ref, hb_ref, wea_ref, web_ref, weg_ref, bev_ref,
                 waf_ref, wab_ref, wtf_ref, wtb_ref, wg_ref, ba_ref,
                 ev_out_ref, ar_out_ref, e1_ref, a2f_ref, a2b_ref):
    hf = hf_ref[...].reshape(L * B, H)
    hb = hb_ref[...].reshape(L * B, H)
    e1_ref[...] = (
        jnp.dot(hf, wea_ref[...], preferred_element_type=_F32)
        + jnp.dot(hb, web_ref[...], preferred_element_type=_F32)
    ).reshape(L, B, NE)
    a2f_ref[...] = jnp.dot(
        hf, wtf_ref[...], preferred_element_type=_F32).reshape(L, B, NA)
    a2b_ref[...] = jnp.dot(
        hb, wtb_ref[...], preferred_element_type=_F32).reshape(L, B, NA)
    a1 = (
        jnp.dot(hf, waf_ref[...], preferred_element_type=_F32)
        + jnp.dot(hb, wab_ref[...], preferred_element_type=_F32)
    )
    bev = bev_ref[...]
    ba = ba_ref[...]
    weg = weg_ref[...]
    wg = wg_ref[...]
    NG = NA - 1 + NE - 1
    iota_ne = lax.broadcasted_iota(jnp.int32, (B, NE), 1)
    iota_na = lax.broadcasted_iota(jnp.int32, (L * B, NA), 1)
    iota_e = lax.broadcasted_iota(jnp.int32, (B, NE - 1), 1)
    iota_g = lax.broadcasted_iota(jnp.int32, (L * B, NG), 1)

    def step(i, carry):
        g_trg, g_cat = carry
        ev = (e1_ref[i]
              + jnp.dot(g_trg, weg, preferred_element_type=_F32)) + bev
        ev_out_ref[i] = ev
        ev_pred = jnp.min(
            jnp.where(ev == jnp.max(ev, axis=1, keepdims=True), iota_ne, NE),
            axis=1, keepdims=True)
        ar = ((((a1
                 + jnp.broadcast_to(a2f_ref[i][None],
                                    (L, B, NA)).reshape(L * B, NA))
                + jnp.broadcast_to(a2b_ref[i][None],
                                   (L, B, NA)).reshape(L * B, NA))
               + jnp.dot(g_cat, wg, preferred_element_type=_F32))
              + ba)
        ar_out_ref[i] = ar.reshape(L, B, NA)
        a_pred = jnp.min(
            jnp.where(ar == jnp.max(ar, axis=1, keepdims=True), iota_na, NA),
            axis=1, keepdims=True)
        ev_mask = ev_pred > 0
        e_idx = jnp.maximum(ev_pred - 1, 0)
        arg_mask = a_pred > 0
        a_idx = jnp.maximum(a_pred - 1, 0)
        evm_f = jnp.broadcast_to(ev_mask[None], (L, B, 1)).reshape(L * B, 1)
        eix_f = jnp.broadcast_to(e_idx[None], (L, B, 1)).reshape(L * B, 1)
        g_trg = jnp.maximum(
            g_trg, jnp.where((iota_e == e_idx) & ev_mask, 1.0, 0.0))
        g_cat = jnp.maximum(
            g_cat,
            jnp.where(((iota_g == a_idx) | (iota_g == eix_f + (NA - 1)))
                      & arg_mask & evm_f, 1.0, 0.0))
        return (g_trg, g_cat)

    init = (jnp.zeros((B, NE - 1), _F32),
            jnp.zeros((L * B, NG), _F32))
    lax.fori_loop(0, L, step, init)


def _run_decode(hf, hb, w_event, b_event, w_arg, b_arg):
    wea = w_event[:, 0:H].T
    web = w_event[:, H:2 * H].T
    weg = w_event[:, 2 * H:2 * H + NE - 1].T
    waf = w_arg[:, 0:H].T
    wab = w_arg[:, H:2 * H].T
    wtf = w_arg[:, 2 * H:3 * H].T
    wtb = w_arg[:, 3 * H:4 * H].T
    wg = w_arg[:, 4 * H:4 * H + NA - 1 + NE - 1].T
    return pl.pallas_call(
        _decode_body,
        out_shape=[
            jax.ShapeDtypeStruct((L, B, NE), _F32),
            jax.ShapeDtypeStruct((L, L, B, NA), _F32),
        ],
        scratch_shapes=[
            pltpu.VMEM((L, B, NE), _F32),
            pltpu.VMEM((L, B, NA), _F32),
            pltpu.VMEM((L, B, NA), _F32),
        ],
    )(hf, hb, wea, web, weg, b_event.reshape(1, NE),
      waf, wab, wtf, wtb, wg, b_arg.reshape(1, NA))


def kernel(input_ids, embedding, W_ih_f, W_hh_f, b_f, W_ih_b, W_hh_b, b_b,
           W_event, b_event, W_arg, b_arg):
    idx = input_ids.astype(jnp.int32).T.reshape(L * B)
    n_pad = 64 * 32
    idx_pad = jnp.concatenate(
        [idx, jnp.zeros((n_pad - L * B,), jnp.int32)])
    rows = _sc_gather(embedding.astype(_F32), idx_pad)
    emb = rows[:L * B].reshape(L, B, D)

    hf, hb = _run_lstm(
        emb, W_ih_f.T, W_hh_f.T, b_f.reshape(1, 4 * H),
        W_ih_b.T, W_hh_b.T, b_b.reshape(1, 4 * H))

    ev, ar = _run_decode(hf, hb, W_event, b_event, W_arg, b_arg)
    event_logits = ev.transpose(1, 0, 2)
    arguments_logits = ar.transpose(2, 0, 1, 3)
    return event_logits, arguments_logits


# speedup vs baseline: 14.8809x; 2.5214x over previous
import functools

import jax
import jax.numpy as jnp
from jax import lax
from jax.experimental import pallas as pl
from jax.experimental.pallas import tpu as pltpu
from jax.experimental.pallas import tpu_sc as plsc

B = 32
L = 50
NE = 34
NA = 36
D = 128
H = 256
NG = NA - 1 + NE - 1
_F32 = jnp.float32


def _sc_gather(table, idx):
    info = plsc.get_sparse_core_info()
    nw = info.num_cores * info.num_subcores
    n = idx.shape[0]
    per = n // nw

    mesh = plsc.VectorSubcoreMesh(core_axis_name="c", subcore_axis_name="s")

    @functools.partial(
        pl.kernel,
        mesh=mesh,
        out_type=jax.ShapeDtypeStruct((n, D), _F32),
        scratch_types=[
            pltpu.VMEM((per,), jnp.int32),
            pltpu.VMEM((per, D), _F32),
            pltpu.SemaphoreType.DMA,
        ],
    )
    def gather_kernel(table_hbm, idx_hbm, out_hbm, idx_v, rows_v, sem):
        wid = lax.axis_index("s") * info.num_cores + lax.axis_index("c")
        base = wid * per
        pltpu.sync_copy(idx_hbm.at[pl.ds(base, per)], idx_v)
        pltpu.async_copy(table_hbm.at[idx_v], rows_v, sem).wait()
        pltpu.sync_copy(rows_v, out_hbm.at[pl.ds(base, per)])

    return gather_kernel(table, idx)


def _lstm_proj_body(emb_ref, wif_ref, whf_ref, bf_ref, wib_ref, whb_ref,
                    bb_ref, wea_ref, web_ref, wtf_ref, wtb_ref, waf_ref,
                    wab_ref, e1_ref, a2f_ref, a2b_ref, a1_ref,
                    xf_ref, xb_ref, hf_ref, hb_ref):
    emb = emb_ref[...].reshape(L * B, D)
    xf_ref[...] = jnp.dot(
        emb, wif_ref[...], preferred_element_type=_F32).reshape(L, B, 4 * H)
    xb_ref[...] = jnp.dot(
        emb, wib_ref[...], preferred_element_type=_F32).reshape(L, B, 4 * H)
    whf = whf_ref[...]
    whb = whb_ref[...]
    bf = bf_ref[...]
    bb = bb_ref[...]
    zero = jnp.zeros((B, H), _F32)

    def cell(gates, c):
        i = jax.nn.sigmoid(gates[:, 0:H])
        f = jax.nn.sigmoid(gates[:, H:2 * H])
        g = jnp.tanh(gates[:, 2 * H:3 * H])
        o = jax.nn.sigmoid(gates[:, 3 * H:4 * H])
        c = f * c + i * g
        return o * jnp.tanh(c), c

    def step(k, carry):
        hfv, cf, hbv, cb = carry
        tb = L - 1 - k
        gf = (xf_ref[k] + jnp.dot(hfv, whf, preferred_element_type=_F32)) + bf
        gb = (xb_ref[tb] + jnp.dot(hbv, whb, preferred_element_type=_F32)) + bb
        hfv, cf = cell(gf, cf)
        hbv, cb = cell(gb, cb)
        hf_ref[k] = hfv
        hb_ref[tb] = hbv
        return (hfv, cf, hbv, cb)

    lax.fori_loop(0, L, step, (zero, zero, zero, zero))

    hf = hf_ref[...].reshape(L * B, H)
    hb = hb_ref[...].reshape(L * B, H)
    e1_ref[...] = (
        jnp.dot(hf, wea_ref[...], preferred_element_type=_F32)
        + jnp.dot(hb, web_ref[...], preferred_element_type=_F32)
    ).reshape(L, B, NE)
    a2f_ref[...] = jnp.dot(
        hf, wtf_ref[...], preferred_element_type=_F32).reshape(L, B, NA)
    a2b_ref[...] = jnp.dot(
        hb, wtb_ref[...], preferred_element_type=_F32).reshape(L, B, NA)
    a1_ref[...] = (
        jnp.dot(hf, waf_ref[...], preferred_element_type=_F32)
        + jnp.dot(hb, wab_ref[...], preferred_element_type=_F32)
    ).reshape(L, B, NA)


def _run_lstm_proj(emb, wif_t, whf_t, bf, wib_t, whb_t, bb,
                   wea, web, wtf, wtb, waf, wab):
    return pl.pallas_call(
        _lstm_proj_body,
        out_shape=[
            jax.ShapeDtypeStruct((L, B, NE), _F32),
            jax.ShapeDtypeStruct((L, B, NA), _F32),
            jax.ShapeDtypeStruct((L, B, NA), _F32),
            jax.ShapeDtypeStruct((L, B, NA), _F32),
        ],
        scratch_shapes=[
            pltpu.VMEM((L, B, 4 * H), _F32),
            pltpu.VMEM((L, B, 4 * H), _F32),
            pltpu.VMEM((L, B, H), _F32),
            pltpu.VMEM((L, B, H), _F32),
        ],
    )(emb, wif_t, whf_t, bf, wib_t, whb_t, bb, wea, web, wtf, wtb, waf, wab)


def _decode_body(e1_ref, a2f_ref, a2b_ref, a1_ref, weg_ref, bev_ref,
                 wg_ref, ba_ref, ev_out_ref, ar_out_ref, gtrg_ref, gcat_ref):
    i = pl.program_id(0)

    @pl.when(i == 0)
    def _init():
        gtrg_ref[...] = jnp.zeros((B, NE - 1), _F32)
        gcat_ref[...] = jnp.zeros((B * L, NG), _F32)

    g_trg = gtrg_ref[...]
    g_cat = gcat_ref[...]

    ev = (e1_ref[i]
          + jnp.dot(g_trg, weg_ref[...], preferred_element_type=_F32)
          ) + bev_ref[...]
    ev_out_ref[:, 0, 0, :] = ev
    iota_ne = lax.broadcasted_iota(jnp.int32, (B, NE), 1)
    ev_pred = jnp.min(
        jnp.where(ev == jnp.max(ev, axis=1, keepdims=True), iota_ne, NE),
        axis=1, keepdims=True)

    def bc(x):
        return jnp.broadcast_to(x[:, None, :], (B, L, NA)).reshape(B * L, NA)

    ar = (((a1_ref[...] + bc(a2f_ref[i])) + bc(a2b_ref[i]))
          + jnp.dot(g_cat, wg_ref[...], preferred_element_type=_F32)
          ) + ba_ref[...]
    ar_out_ref[:, 0, :, :] = ar.reshape(B, L, NA)
    iota_na = lax.broadcasted_iota(jnp.int32, (B * L, NA), 1)
    a_pred = jnp.min(
        jnp.where(ar == jnp.max(ar, axis=1, keepdims=True), iota_na, NA),
        axis=1, keepdims=True)

    ev_mask = ev_pred > 0
    e_idx = jnp.maximum(ev_pred - 1, 0)
    arg_mask = a_pred > 0
    a_idx = jnp.maximum(a_pred - 1, 0)
    evm_f = jnp.broadcast_to(ev_mask[:, None, :], (B, L, 1)).reshape(B * L, 1)
    eix_f = jnp.broadcast_to(e_idx[:, None, :], (B, L, 1)).reshape(B * L, 1)
    iota_e = lax.broadcasted_iota(jnp.int32, (B, NE - 1), 1)
    iota_g = lax.broadcasted_iota(jnp.int32, (B * L, NG), 1)
    gtrg_ref[...] = jnp.maximum(
        g_trg, jnp.where((iota_e == e_idx) & ev_mask, 1.0, 0.0))
    gcat_ref[...] = jnp.maximum(
        g_cat,
        jnp.where(((iota_g == a_idx) | (iota_g == eix_f + (NA - 1)))
                  & arg_mask & evm_f, 1.0, 0.0))


def _run_decode(e1, a2f, a2b, a1b, w_event, b_event, w_arg, b_arg):
    weg = w_event[:, 2 * H:2 * H + NE - 1].T
    wg = w_arg[:, 4 * H:4 * H + NG].T
    return pl.pallas_call(
        _decode_body,
        grid=(L,),
        in_specs=[
            pl.BlockSpec((L, B, NE), lambda i: (0, 0, 0)),
            pl.BlockSpec((L, B, NA), lambda i: (0, 0, 0)),
            pl.BlockSpec((L, B, NA), lambda i: (0, 0, 0)),
            pl.BlockSpec((B * L, NA), lambda i: (0, 0)),
            pl.BlockSpec((NE - 1, NE), lambda i: (0, 0)),
            pl.BlockSpec((1, NE), lambda i: (0, 0)),
            pl.BlockSpec((NG, NA), lambda i: (0, 0)),
            pl.BlockSpec((1, NA), lambda i: (0, 0)),
        ],
        out_specs=[
            pl.BlockSpec((B, 1, 1, NE), lambda i: (0, i, 0, 0)),
            pl.BlockSpec((B, 1, L, NA), lambda i: (0, i, 0, 0)),
        ],
        out_shape=[
            jax.ShapeDtypeStruct((B, L, 1, NE), _F32),
            jax.ShapeDtypeStruct((B, L, L, NA), _F32),
        ],
        scratch_shapes=[
            pltpu.VMEM((B, NE - 1), _F32),
            pltpu.VMEM((B * L, NG), _F32),
        ],
    )(e1, a2f, a2b, a1b, weg, b_event.reshape(1, NE), wg,
      b_arg.reshape(1, NA))


def kernel(input_ids, embedding, W_ih_f, W_hh_f, b_f, W_ih_b, W_hh_b, b_b,
           W_event, b_event, W_arg, b_arg):
    idx = input_ids.astype(jnp.int32).T.reshape(L * B)
    n_pad = 64 * 32
    idx_pad = jnp.concatenate(
        [idx, jnp.zeros((n_pad - L * B,), jnp.int32)])
    rows = _sc_gather(embedding.astype(_F32), idx_pad)
    emb = rows[:L * B].reshape(L, B, D)

    wea = W_event[:, 0:H].T
    web = W_event[:, H:2 * H].T
    waf = W_arg[:, 0:H].T
    wab = W_arg[:, H:2 * H].T
    wtf = W_arg[:, 2 * H:3 * H].T
    wtb = W_arg[:, 3 * H:4 * H].T
    e1, a2f, a2b, a1 = _run_lstm_proj(
        emb, W_ih_f.T, W_hh_f.T, b_f.reshape(1, 4 * H),
        W_ih_b.T, W_hh_b.T, b_b.reshape(1, 4 * H),
        wea, web, wtf, wtb, waf, wab)
    a1b = a1.transpose(1, 0, 2).reshape(B * L, NA)

    event_logits, arguments_logits = _run_decode(
        e1, a2f, a2b, a1b, W_event, b_event, W_arg, b_arg)
    return event_logits.reshape(B, L, NE), arguments_logits


# speedup vs baseline: 15.0635x; 1.0123x over previous
import functools

import jax
import jax.numpy as jnp
from jax import lax
from jax.experimental import pallas as pl
from jax.experimental.pallas import tpu as pltpu
from jax.experimental.pallas import tpu_sc as plsc

B = 32
L = 50
NE = 34
NA = 36
D = 128
H = 256
NG = NA - 1 + NE - 1
_F32 = jnp.float32


def _sc_gather(table, idx):
    info = plsc.get_sparse_core_info()
    nw = info.num_cores * info.num_subcores
    n = idx.shape[0]
    per = n // nw

    mesh = plsc.VectorSubcoreMesh(core_axis_name="c", subcore_axis_name="s")

    @functools.partial(
        pl.kernel,
        mesh=mesh,
        out_type=jax.ShapeDtypeStruct((n, D), _F32),
        scratch_types=[
            pltpu.VMEM((per,), jnp.int32),
            pltpu.VMEM((per, D), _F32),
            pltpu.SemaphoreType.DMA,
        ],
    )
    def gather_kernel(table_hbm, idx_hbm, out_hbm, idx_v, rows_v, sem):
        wid = lax.axis_index("s") * info.num_cores + lax.axis_index("c")
        base = wid * per
        pltpu.sync_copy(idx_hbm.at[pl.ds(base, per)], idx_v)
        pltpu.async_copy(table_hbm.at[idx_v], rows_v, sem).wait()
        pltpu.sync_copy(rows_v, out_hbm.at[pl.ds(base, per)])

    return gather_kernel(table, idx)


def _lstm_proj_body(emb_ref, wif_ref, whf_ref, bf_ref, wib_ref, whb_ref,
                    bb_ref, wea_ref, web_ref, wtf_ref, wtb_ref, waf_ref,
                    wab_ref, e1_ref, a2f_ref, a2b_ref, a1_ref,
                    xf_ref, xb_ref, hf_ref, hb_ref):
    emb = emb_ref[...].reshape(L * B, D)
    xf_ref[...] = jnp.dot(
        emb, wif_ref[...], preferred_element_type=_F32).reshape(L, B, 4 * H)
    xb_ref[...] = jnp.dot(
        emb, wib_ref[...], preferred_element_type=_F32).reshape(L, B, 4 * H)
    whf = whf_ref[...]
    whb = whb_ref[...]
    bf = bf_ref[...]
    bb = bb_ref[...]
    zero = jnp.zeros((B, H), _F32)

    def cell(gates, c):
        i = jax.nn.sigmoid(gates[:, 0:H])
        f = jax.nn.sigmoid(gates[:, H:2 * H])
        g = jnp.tanh(gates[:, 2 * H:3 * H])
        o = jax.nn.sigmoid(gates[:, 3 * H:4 * H])
        c = f * c + i * g
        return o * jnp.tanh(c), c

    def step(k, carry):
        hfv, cf, hbv, cb = carry
        tb = L - 1 - k
        gf = (xf_ref[k] + jnp.dot(hfv, whf, preferred_element_type=_F32)) + bf
        gb = (xb_ref[tb] + jnp.dot(hbv, whb, preferred_element_type=_F32)) + bb
        hfv, cf = cell(gf, cf)
        hbv, cb = cell(gb, cb)
        hf_ref[k] = hfv
        hb_ref[tb] = hbv
        return (hfv, cf, hbv, cb)

    lax.fori_loop(0, L, step, (zero, zero, zero, zero))

    hf = hf_ref[...].reshape(L * B, H)
    hb = hb_ref[...].reshape(L * B, H)
    e1_ref[...] = (
        jnp.dot(hf, wea_ref[...], preferred_element_type=_F32)
        + jnp.dot(hb, web_ref[...], preferred_element_type=_F32)
    ).reshape(L, B, NE)
    a2f_ref[...] = jnp.dot(
        hf, wtf_ref[...], preferred_element_type=_F32).reshape(L, B, NA)
    a2b_ref[...] = jnp.dot(
        hb, wtb_ref[...], preferred_element_type=_F32).reshape(L, B, NA)
    a1_ref[...] = (
        jnp.dot(hf, waf_ref[...], preferred_element_type=_F32)
        + jnp.dot(hb, wab_ref[...], preferred_element_type=_F32)
    ).reshape(L, B, NA)


def _run_lstm_proj(emb, wif_t, whf_t, bf, wib_t, whb_t, bb,
                   wea, web, wtf, wtb, waf, wab):
    return pl.pallas_call(
        _lstm_proj_body,
        out_shape=[
            jax.ShapeDtypeStruct((L, B, NE), _F32),
            jax.ShapeDtypeStruct((L, B, NA), _F32),
            jax.ShapeDtypeStruct((L, B, NA), _F32),
            jax.ShapeDtypeStruct((L, B, NA), _F32),
        ],
        scratch_shapes=[
            pltpu.VMEM((L, B, 4 * H), _F32),
            pltpu.VMEM((L, B, 4 * H), _F32),
            pltpu.VMEM((L, B, H), _F32),
            pltpu.VMEM((L, B, H), _F32),
        ],
    )(emb, wif_t, whf_t, bf, wib_t, whb_t, bb, wea, web, wtf, wtb, waf, wab)


def _decode_body(e1_ref, a2f_ref, a2b_ref, a1_ref, weg_ref, bev_ref,
                 wg_ref, ba_ref, ev_out_ref, ar_out_ref, gtrg_ref, gcat_ref,
                 a1b_ref):
    i = pl.program_id(0)

    @pl.when(i == 0)
    def _init():
        gtrg_ref[...] = jnp.zeros((B, NE - 1), _F32)
        gcat_ref[...] = jnp.zeros((B * L, NG), _F32)
        for b in range(B):
            a1b_ref[pl.ds(b * L, L), :] = a1_ref[:, b, :]

    g_trg = gtrg_ref[...]
    g_cat = gcat_ref[...]

    ev = (e1_ref[i]
          + jnp.dot(g_trg, weg_ref[...], preferred_element_type=_F32)
          ) + bev_ref[...]
    ev_out_ref[:, 0, 0, :] = ev
    iota_ne = lax.broadcasted_iota(jnp.int32, (B, NE), 1)
    ev_pred = jnp.min(
        jnp.where(ev == jnp.max(ev, axis=1, keepdims=True), iota_ne, NE),
        axis=1, keepdims=True)

    def bc(x):
        return jnp.broadcast_to(x[:, None, :], (B, L, NA)).reshape(B * L, NA)

    ar = (((a1b_ref[...] + bc(a2f_ref[i])) + bc(a2b_ref[i]))
          + jnp.dot(g_cat, wg_ref[...], preferred_element_type=_F32)
          ) + ba_ref[...]
    ar_out_ref[:, 0, :, :] = ar.reshape(B, L, NA)
    iota_na = lax.broadcasted_iota(jnp.int32, (B * L, NA), 1)
    a_pred = jnp.min(
        jnp.where(ar == jnp.max(ar, axis=1, keepdims=True), iota_na, NA),
        axis=1, keepdims=True)

    ev_mask = ev_pred > 0
    e_idx = jnp.maximum(ev_pred - 1, 0)
    arg_mask = a_pred > 0
    a_idx = jnp.maximum(a_pred - 1, 0)
    evm_f = jnp.broadcast_to(ev_mask[:, None, :], (B, L, 1)).reshape(B * L, 1)
    eix_f = jnp.broadcast_to(e_idx[:, None, :], (B, L, 1)).reshape(B * L, 1)
    iota_e = lax.broadcasted_iota(jnp.int32, (B, NE - 1), 1)
    iota_g = lax.broadcasted_iota(jnp.int32, (B * L, NG), 1)
    gtrg_ref[...] = jnp.maximum(
        g_trg, jnp.where((iota_e == e_idx) & ev_mask, 1.0, 0.0))
    gcat_ref[...] = jnp.maximum(
        g_cat,
        jnp.where(((iota_g == a_idx) | (iota_g == eix_f + (NA - 1)))
                  & arg_mask & evm_f, 1.0, 0.0))


def _run_decode(e1, a2f, a2b, a1, w_event, b_event, w_arg, b_arg):
    weg = w_event[:, 2 * H:2 * H + NE - 1].T
    wg = w_arg[:, 4 * H:4 * H + NG].T
    return pl.pallas_call(
        _decode_body,
        grid=(L,),
        in_specs=[
            pl.BlockSpec((L, B, NE), lambda i: (0, 0, 0)),
            pl.BlockSpec((L, B, NA), lambda i: (0, 0, 0)),
            pl.BlockSpec((L, B, NA), lambda i: (0, 0, 0)),
            pl.BlockSpec((L, B, NA), lambda i: (0, 0, 0)),
            pl.BlockSpec((NE - 1, NE), lambda i: (0, 0)),
            pl.BlockSpec((1, NE), lambda i: (0, 0)),
            pl.BlockSpec((NG, NA), lambda i: (0, 0)),
            pl.BlockSpec((1, NA), lambda i: (0, 0)),
        ],
        out_specs=[
            pl.BlockSpec((B, 1, 1, NE), lambda i: (0, i, 0, 0)),
            pl.BlockSpec((B, 1, L, NA), lambda i: (0, i, 0, 0)),
        ],
        out_shape=[
            jax.ShapeDtypeStruct((B, L, 1, NE), _F32),
            jax.ShapeDtypeStruct((B, L, L, NA), _F32),
        ],
        scratch_shapes=[
            pltpu.VMEM((B, NE - 1), _F32),
            pltpu.VMEM((B * L, NG), _F32),
            pltpu.VMEM((B * L, NA), _F32),
        ],
    )(e1, a2f, a2b, a1, weg, b_event.reshape(1, NE), wg,
      b_arg.reshape(1, NA))


def kernel(input_ids, embedding, W_ih_f, W_hh_f, b_f, W_ih_b, W_hh_b, b_b,
           W_event, b_event, W_arg, b_arg):
    idx = input_ids.astype(jnp.int32).T.reshape(L * B)
    n_pad = 64 * 32
    idx_pad = jnp.concatenate(
        [idx, jnp.zeros((n_pad - L * B,), jnp.int32)])
    rows = _sc_gather(embedding.astype(_F32), idx_pad)
    emb = rows[:L * B].reshape(L, B, D)

    wea = W_event[:, 0:H].T
    web = W_event[:, H:2 * H].T
    waf = W_arg[:, 0:H].T
    wab = W_arg[:, H:2 * H].T
    wtf = W_arg[:, 2 * H:3 * H].T
    wtb = W_arg[:, 3 * H:4 * H].T
    e1, a2f, a2b, a1 = _run_lstm_proj(
        emb, W_ih_f.T, W_hh_f.T, b_f.reshape(1, 4 * H),
        W_ih_b.T, W_hh_b.T, b_b.reshape(1, 4 * H),
        wea, web, wtf, wtb, waf, wab)
    event_logits, arguments_logits = _run_decode(
        e1, a2f, a2b, a1, W_event, b_event, W_arg, b_arg)
    return event_logits.reshape(B, L, NE), arguments_logits


# speedup vs baseline: 15.1377x; 1.0049x over previous
import functools

import jax
import jax.numpy as jnp
from jax import lax
from jax.experimental import pallas as pl
from jax.experimental.pallas import tpu as pltpu
from jax.experimental.pallas import tpu_sc as plsc

B = 32
L = 50
NE = 34
NA = 36
D = 128
H = 256
NG = NA - 1 + NE - 1
_F32 = jnp.float32


def _sc_gather(table, idx):
    info = plsc.get_sparse_core_info()
    nw = info.num_cores * info.num_subcores
    n = idx.shape[0]
    per = n // nw

    mesh = plsc.VectorSubcoreMesh(core_axis_name="c", subcore_axis_name="s")

    @functools.partial(
        pl.kernel,
        mesh=mesh,
        out_type=jax.ShapeDtypeStruct((n, D), _F32),
        scratch_types=[
            pltpu.VMEM((per,), jnp.int32),
            pltpu.VMEM((per, D), _F32),
            pltpu.SemaphoreType.DMA,
        ],
    )
    def gather_kernel(table_hbm, idx_hbm, out_hbm, idx_v, rows_v, sem):
        wid = lax.axis_index("s") * info.num_cores + lax.axis_index("c")
        base = wid * per
        pltpu.sync_copy(idx_hbm.at[pl.ds(base, per)], idx_v)
        pltpu.async_copy(table_hbm.at[idx_v], rows_v, sem).wait()
        pltpu.sync_copy(rows_v, out_hbm.at[pl.ds(base, per)])

    return gather_kernel(table, idx)


def _fused_body(emb_ref, wif_ref, whf_ref, bf_ref, wib_ref, whb_ref, bb_ref,
                wea_ref, web_ref, wtf_ref, wtb_ref, waf_ref, wab_ref,
                weg_ref, bev_ref, wg_ref, ba_ref,
                ev_out_ref, ar_out_ref,
                xf_ref, xb_ref, hf_ref, hb_ref,
                e1_ref, a2f_ref, a2b_ref, a1b_ref, gtrg_ref, gcat_ref):
    i = pl.program_id(0)

    @pl.when(i == 0)
    def _pre():
        emb = emb_ref[...].reshape(L * B, D)
        xf_ref[...] = jnp.dot(
            emb, wif_ref[...],
            preferred_element_type=_F32).reshape(L, B, 4 * H)
        xb_ref[...] = jnp.dot(
            emb, wib_ref[...],
            preferred_element_type=_F32).reshape(L, B, 4 * H)
        whf = whf_ref[...]
        whb = whb_ref[...]
        bf = bf_ref[...]
        bb = bb_ref[...]
        zero = jnp.zeros((B, H), _F32)

        def cell(gates, c):
            ig = jax.nn.sigmoid(gates[:, 0:H])
            f = jax.nn.sigmoid(gates[:, H:2 * H])
            g = jnp.tanh(gates[:, 2 * H:3 * H])
            o = jax.nn.sigmoid(gates[:, 3 * H:4 * H])
            c = f * c + ig * g
            return o * jnp.tanh(c), c

        def step(k, carry):
            hfv, cf, hbv, cb = carry
            tb = L - 1 - k
            gf = (xf_ref[k]
                  + jnp.dot(hfv, whf, preferred_element_type=_F32)) + bf
            gb = (xb_ref[tb]
                  + jnp.dot(hbv, whb, preferred_element_type=_F32)) + bb
            hfv, cf = cell(gf, cf)
            hbv, cb = cell(gb, cb)
            hf_ref[k] = hfv
            hb_ref[tb] = hbv
            return (hfv, cf, hbv, cb)

        lax.fori_loop(0, L, step, (zero, zero, zero, zero))

        hf = hf_ref[...].reshape(L * B, H)
        hb = hb_ref[...].reshape(L * B, H)
        e1_ref[...] = (
            jnp.dot(hf, wea_ref[...], preferred_element_type=_F32)
            + jnp.dot(hb, web_ref[...], preferred_element_type=_F32)
        ).reshape(L, B, NE)
        a2f_ref[...] = jnp.dot(
            hf, wtf_ref[...], preferred_element_type=_F32).reshape(L, B, NA)
        a2b_ref[...] = jnp.dot(
            hb, wtb_ref[...], preferred_element_type=_F32).reshape(L, B, NA)
        a1 = (
            jnp.dot(hf, waf_ref[...], preferred_element_type=_F32)
            + jnp.dot(hb, wab_ref[...], preferred_element_type=_F32)
        ).reshape(L, B, NA)
        for b in range(B):
            a1b_ref[pl.ds(b * L, L), :] = a1[:, b, :]
        gtrg_ref[...] = jnp.zeros((B, NE - 1), _F32)
        gcat_ref[...] = jnp.zeros((B * L, NG), _F32)

    @pl.when(i > 0)
    def _step():
        t = i - 1
        g_trg = gtrg_ref[...]
        g_cat = gcat_ref[...]

        ev = (e1_ref[t]
              + jnp.dot(g_trg, weg_ref[...], preferred_element_type=_F32)
              ) + bev_ref[...]
        ev_out_ref[:, 0, 0, :] = ev
        iota_ne = lax.broadcasted_iota(jnp.int32, (B, NE), 1)
        ev_pred = jnp.min(
            jnp.where(ev == jnp.max(ev, axis=1, keepdims=True), iota_ne, NE),
            axis=1, keepdims=True)

        def bc(x):
            return jnp.broadcast_to(
                x[:, None, :], (B, L, NA)).reshape(B * L, NA)

        ar = (((a1b_ref[...] + bc(a2f_ref[t])) + bc(a2b_ref[t]))
              + jnp.dot(g_cat, wg_ref[...], preferred_element_type=_F32)
              ) + ba_ref[...]
        ar_out_ref[:, 0, :, :] = ar.reshape(B, L, NA)
        iota_na = lax.broadcasted_iota(jnp.int32, (B * L, NA), 1)
        a_pred = jnp.min(
            jnp.where(ar == jnp.max(ar, axis=1, keepdims=True), iota_na, NA),
            axis=1, keepdims=True)

        ev_mask = ev_pred > 0
        e_idx = jnp.maximum(ev_pred - 1, 0)
        arg_mask = a_pred > 0
        a_idx = jnp.maximum(a_pred - 1, 0)
        evm_f = jnp.broadcast_to(
            ev_mask[:, None, :], (B, L, 1)).reshape(B * L, 1)
        eix_f = jnp.broadcast_to(
            e_idx[:, None, :], (B, L, 1)).reshape(B * L, 1)
        iota_e = lax.broadcasted_iota(jnp.int32, (B, NE - 1), 1)
        iota_g = lax.broadcasted_iota(jnp.int32, (B * L, NG), 1)
        gtrg_ref[...] = jnp.maximum(
            g_trg, jnp.where((iota_e == e_idx) & ev_mask, 1.0, 0.0))
        gcat_ref[...] = jnp.maximum(
            g_cat,
            jnp.where(((iota_g == a_idx) | (iota_g == eix_f + (NA - 1)))
                      & arg_mask & evm_f, 1.0, 0.0))


def _run_fused(emb, wif_t, whf_t, bf, wib_t, whb_t, bb,
               wea, web, wtf, wtb, waf, wab, w_event, b_event, w_arg, b_arg):
    weg = w_event[:, 2 * H:2 * H + NE - 1].T
    wg = w_arg[:, 4 * H:4 * H + NG].T
    def whole(x):
        nd = len(x.shape)
        return pl.BlockSpec(x.shape, lambda i, _n=nd: (0,) * _n)

    shift = lambda i: (0, jnp.maximum(i - 1, 0), 0, 0)
    ins = [emb, wif_t, whf_t, bf, wib_t, whb_t, bb,
           wea, web, wtf, wtb, waf, wab,
           weg, b_event.reshape(1, NE), wg, b_arg.reshape(1, NA)]
    return pl.pallas_call(
        _fused_body,
        grid=(L + 1,),
        in_specs=[whole(x) for x in ins],
        out_specs=[
            pl.BlockSpec((B, 1, 1, NE), shift),
            pl.BlockSpec((B, 1, L, NA), shift),
        ],
        out_shape=[
            jax.ShapeDtypeStruct((B, L, 1, NE), _F32),
            jax.ShapeDtypeStruct((B, L, L, NA), _F32),
        ],
        scratch_shapes=[
            pltpu.VMEM((L, B, 4 * H), _F32),
            pltpu.VMEM((L, B, 4 * H), _F32),
            pltpu.VMEM((L, B, H), _F32),
            pltpu.VMEM((L, B, H), _F32),
            pltpu.VMEM((L, B, NE), _F32),
            pltpu.VMEM((L, B, NA), _F32),
            pltpu.VMEM((L, B, NA), _F32),
            pltpu.VMEM((B * L, NA), _F32),
            pltpu.VMEM((B, NE - 1), _F32),
            pltpu.VMEM((B * L, NG), _F32),
        ],
    )(*ins)


def kernel(input_ids, embedding, W_ih_f, W_hh_f, b_f, W_ih_b, W_hh_b, b_b,
           W_event, b_event, W_arg, b_arg):
    idx = input_ids.astype(jnp.int32).T.reshape(L * B)
    n_pad = 64 * 32
    idx_pad = jnp.concatenate(
        [idx, jnp.zeros((n_pad - L * B,), jnp.int32)])
    rows = _sc_gather(embedding.astype(_F32), idx_pad)
    emb = rows[:L * B].reshape(L, B, D)

    ev, ar = _run_fused(
        emb, W_ih_f.T, W_hh_f.T, b_f.reshape(1, 4 * H),
        W_ih_b.T, W_hh_b.T, b_b.reshape(1, 4 * H),
        W_event[:, 0:H].T, W_event[:, H:2 * H].T,
        W_arg[:, 2 * H:3 * H].T, W_arg[:, 3 * H:4 * H].T,
        W_arg[:, 0:H].T, W_arg[:, H:2 * H].T,
        W_event, b_event, W_arg, b_arg)
    return ev.reshape(B, L, NE), ar
